# baseline (speedup 1.0000x reference)
import functools

import jax
import jax.numpy as jnp
from jax import lax
from jax.experimental import pallas as pl
from jax.experimental.pallas import tpu as pltpu
from jax.experimental.pallas import tpu_sc as plsc

_D = 32
_T = 200
_NC = 2
_NS = 16
_NW = _NC * _NS
_RI = 2
_NBUF = 4
_SEG = ((0, 128), (128, _T - 128))


def _gather_body(idx_hbm, table_hbm, out_hbm, *scratch):
    idx_v, rows_v = scratch[0], scratch[1]
    sem_idx = scratch[2:2 + _NBUF]
    sem_gat = scratch[2 + _NBUF:2 + 2 * _NBUF]
    sem_out = scratch[2 + 2 * _NBUF:2 + 3 * _NBUF]

    wid = lax.axis_index("s") * _NC + lax.axis_index("c")
    rows_per_w = out_hbm.shape[0] // _NW
    n_chunks = rows_per_w // _RI
    base = wid * rows_per_w

    def start_idx(c, b):
        pltpu.async_copy(
            idx_hbm.at[pl.ds(base + c * _RI, _RI)], idx_v.at[b], sem_idx[b])

    def wait_idx(b):
        pltpu.make_async_copy(
            idx_hbm.at[pl.ds(0, _RI)], idx_v.at[b], sem_idx[b]).wait()

    def wait_gat(b):
        pltpu.make_async_copy(
            out_hbm.at[pl.ds(0, _RI)], rows_v.at[b], sem_gat[b]).wait()

    def start_out(c, b):
        pltpu.async_copy(
            rows_v.at[b], out_hbm.at[pl.ds(base + c * _RI, _RI)], sem_out[b])

    def wait_out(b):
        pltpu.make_async_copy(
            rows_v.at[b], out_hbm.at[pl.ds(0, _RI)], sem_out[b]).wait()

    def fire_gathers(b):
        for i in range(_RI):
            for off, ln in _SEG:
                pltpu.async_copy(
                    table_hbm.at[idx_v.at[b, i, pl.ds(off, ln)]],
                    rows_v.at[b, i, pl.ds(off, ln)],
                    sem_gat[b])

    for b in range(_NBUF):
        start_idx(b, b)

    def outer(g, carry):
        for b in range(_NBUF):
            c = g * _NBUF + b
            wait_idx(b)

            @pl.when(g > 0)
            def _():
                wait_out(b)

            fire_gathers(b)

            bp = (b - 1) % _NBUF

            @pl.when(c > 0)
            def _():
                wait_gat(bp)
                start_out(c - 1, bp)

                @pl.when(c - 1 + _NBUF < n_chunks)
                def _():
                    start_idx(c - 1 + _NBUF, bp)
        return carry

    lax.fori_loop(0, n_chunks // _NBUF, outer, 0)

    bl = (n_chunks - 1) % _NBUF
    wait_gat(bl)
    start_out(n_chunks - 1, bl)
    for b in range(_NBUF):
        wait_out(b)


@jax.jit
def _run(idx2d, table):
    n_rows = idx2d.shape[0]
    mesh = plsc.VectorSubcoreMesh(core_axis_name="c", subcore_axis_name="s")
    scratch = [
        pltpu.VMEM((_NBUF, _RI, _T), jnp.int32),
        pltpu.VMEM((_NBUF, _RI, _T, _D), jnp.float32),
    ] + [pltpu.SemaphoreType.DMA] * (3 * _NBUF)
    return pl.kernel(
        _gather_body,
        mesh=mesh,
        out_type=jax.ShapeDtypeStruct((n_rows, _T, _D), jnp.float32),
        scratch_types=scratch,
        compiler_params=pltpu.CompilerParams(use_tc_tiling_on_sc=False),
    )(idx2d, table)


def kernel(time_steps, table):
    return _run(time_steps.astype(jnp.int32), table)
